# baseline (speedup 1.0000x reference)
import functools

import jax
import jax.numpy as jnp
from jax import lax
from jax.experimental import pallas as pl
from jax.experimental.pallas import tpu as pltpu
from jax.experimental.pallas import tpu_sc as plsc

N_NODES = 100000
IN_CH = 10
HIDDEN = 16

NC = 2
NS = 16
NW = NC * NS
LANE = 128
K = 6
G = 262
EDGES_PAD = NW * G * K * LANE
NROWS = NW * G * K
ROWS_PER_TILE = 6400
N_PAD = NS * ROWS_PER_TILE
TRASH = N_NODES
OCHUNK = 640


def _sc_segment_sum(e3d, x_pad):
    mesh = plsc.VectorSubcoreMesh(core_axis_name="c", subcore_axis_name="s")

    @functools.partial(
        pl.kernel,
        out_type=jax.ShapeDtypeStruct((NC, N_PAD, HIDDEN), jnp.float32),
        mesh=mesh,
        compiler_params=pltpu.CompilerParams(use_tc_tiling_on_sc=False),
        scratch_types=[
            pltpu.VMEM_SHARED((N_PAD, HIDDEN), jnp.float32),
            pltpu.VMEM((K, LANE), jnp.int32),
            pltpu.VMEM((K, LANE), jnp.int32),
            pltpu.VMEM((K, LANE), jnp.int32),
            pltpu.VMEM((K, LANE), jnp.int32),
            pltpu.VMEM((K * LANE, HIDDEN), jnp.float32),
            pltpu.VMEM((K * LANE, HIDDEN), jnp.float32),
            pltpu.SemaphoreType.DMA((2,)),
            pltpu.SemaphoreType.DMA((2,)),
            pltpu.SemaphoreType.DMA((2,)),
        ],
    )
    def kern(e_hbm, x_hbm, out_hbm,
             acc, src0, src1, dst0, dst1, rows0, rows1, isem, gsem, ssem):
        src_hbm = e_hbm.at[0]
        dst_hbm = e_hbm.at[1]
        cid = lax.axis_index("c")
        sid = lax.axis_index("s")
        wid = cid * NS + sid

        srcb = [src0, src1]
        dstb = [dst0, dst1]
        rowsb = [rows0, rows1]

        zero16 = jnp.zeros((HIDDEN,), jnp.float32)

        @pl.loop(0, LANE)
        def _(i):
            rows0[i, :] = zero16

        zbase = sid * ROWS_PER_TILE

        @pl.loop(0, ROWS_PER_TILE // LANE)
        def _(b):
            pltpu.sync_copy(rows0.at[pl.ds(0, LANE), :],
                            acc.at[pl.ds(zbase + b * LANE, LANE), :])

        plsc.subcore_barrier()

        row0 = wid * (G * K)

        def idx_start(h, b):
            r = row0 + h * K
            pltpu.async_copy(src_hbm.at[pl.ds(r, K)], srcb[b], isem.at[b])
            pltpu.async_copy(dst_hbm.at[pl.ds(r, K)], dstb[b], isem.at[b])

        def idx_wait(b):
            pltpu.make_async_copy(src_hbm.at[pl.ds(0, K)], srcb[b],
                                  isem.at[b]).wait()
            pltpu.make_async_copy(dst_hbm.at[pl.ds(0, K)], dstb[b],
                                  isem.at[b]).wait()

        def gather_start(b):
            for j in range(K):
                pltpu.async_copy(x_hbm.at[srcb[b].at[j]],
                                 rowsb[b].at[pl.ds(j * LANE, LANE), :],
                                 gsem.at[b])

        def gather_wait(b):
            for j in range(K):
                pltpu.make_async_copy(x_hbm.at[srcb[b].at[j]],
                                      rowsb[b].at[pl.ds(j * LANE, LANE), :],
                                      gsem.at[b]).wait()

        def scatter_start(b):
            for j in range(K):
                pltpu.async_copy(rowsb[b].at[pl.ds(j * LANE, LANE), :],
                                 acc.at[dstb[b].at[j]], ssem.at[b], add=True)

        def scatter_wait(b):
            for j in range(K):
                pltpu.make_async_copy(rowsb[b].at[pl.ds(j * LANE, LANE), :],
                                      acc.at[dstb[b].at[j]],
                                      ssem.at[b]).wait()

        idx_start(0, 0)

        @pl.loop(0, G // 2)
        def _(p):
            g = 2 * p
            idx_wait(0)
            gather_start(0)

            @pl.when(p > 0)
            def _():
                scatter_wait(1)

            idx_start(g + 1, 1)
            gather_wait(0)
            scatter_start(0)

            idx_wait(1)
            gather_start(1)
            scatter_wait(0)

            @pl.when(p + 1 < G // 2)
            def _():
                idx_start(g + 2, 0)

            gather_wait(1)
            scatter_start(1)

        scatter_wait(1)

        plsc.subcore_barrier()

        @pl.loop(0, ROWS_PER_TILE // OCHUNK)
        def _(b):
            base = zbase + b * OCHUNK
            pltpu.sync_copy(acc.at[pl.ds(base, OCHUNK), :],
                            rows0.at[pl.ds(0, OCHUNK), :])
            pltpu.sync_copy(rows0.at[pl.ds(0, OCHUNK), :],
                            out_hbm.at[cid, pl.ds(base, OCHUNK), :])

    return kern(e3d, x_pad)


def _tc_combine(partials, x, W_l, W_r, b_l2d):
    B = 4000
    grid = (N_NODES // B,)

    def body(p_ref, x_ref, wl_ref, wr_ref, b_ref, o_ref):
        s = p_ref[0] + p_ref[1]
        cnt = jnp.clip(s[:, IN_CH:IN_CH + 1], 1.0, None)
        mean = s[:, :IN_CH] / cnt
        o_ref[...] = (
            jnp.dot(mean, wl_ref[...], preferred_element_type=jnp.float32)
            + jnp.dot(x_ref[...], wr_ref[...], preferred_element_type=jnp.float32)
            + b_ref[...]
        )

    return pl.pallas_call(
        body,
        grid=grid,
        in_specs=[
            pl.BlockSpec((NC, B, HIDDEN), lambda i: (0, i, 0)),
            pl.BlockSpec((B, IN_CH), lambda i: (i, 0)),
            pl.BlockSpec((IN_CH, HIDDEN), lambda i: (0, 0)),
            pl.BlockSpec((IN_CH, HIDDEN), lambda i: (0, 0)),
            pl.BlockSpec((1, HIDDEN), lambda i: (0, 0)),
        ],
        out_specs=pl.BlockSpec((B, HIDDEN), lambda i: (i, 0)),
        out_shape=jax.ShapeDtypeStruct((N_NODES, HIDDEN), jnp.float32),
    )(partials, x, W_l, W_r, b_l2d)


@jax.jit
def kernel(x, edge_index, W_l, W_r, b_l):
    idx = edge_index.astype(jnp.int32)
    n_edges = idx.shape[1]
    pad = EDGES_PAD - n_edges
    trash_rows = TRASH + jnp.arange(pad, dtype=jnp.int32) % (N_PAD - TRASH)
    e_pad = jnp.concatenate(
        [idx, jnp.broadcast_to(trash_rows, (2, pad))], axis=1)
    e3d = e_pad.reshape(2, NROWS, LANE)

    x_pad = jnp.concatenate(
        [x, jnp.ones((N_NODES, 1), jnp.float32),
         jnp.zeros((N_NODES, HIDDEN - IN_CH - 1), jnp.float32)], axis=1)
    x_pad = jnp.pad(x_pad, ((0, N_PAD - N_NODES), (0, 0)))

    partials = _sc_segment_sum(e3d, x_pad)
    return _tc_combine(partials, x, W_l, W_r, b_l.reshape(1, HIDDEN))


# speedup vs baseline: 1.1488x; 1.1488x over previous
import functools

import jax
import jax.numpy as jnp
from jax import lax
from jax.experimental import pallas as pl
from jax.experimental.pallas import tpu as pltpu
from jax.experimental.pallas import tpu_sc as plsc

N_NODES = 100000
IN_CH = 10
HIDDEN = 16

NC = 2
NS = 16
NW = NC * NS
LANE = 128
K = 6
G = 262
EDGES_PAD = NW * G * K * LANE
NROWS = NW * G * K
ROWS_PER_TILE = 6400
N_PAD = NS * ROWS_PER_TILE
TRASH = N_NODES
OCHUNK = 640


def _sc_segment_sum(e3d, x_pad):
    mesh = plsc.VectorSubcoreMesh(core_axis_name="c", subcore_axis_name="s")

    @functools.partial(
        pl.kernel,
        out_type=jax.ShapeDtypeStruct((NC, N_PAD, HIDDEN), jnp.float32),
        mesh=mesh,
        compiler_params=pltpu.CompilerParams(use_tc_tiling_on_sc=False),
        scratch_types=[
            pltpu.VMEM_SHARED((N_PAD, HIDDEN), jnp.float32),
            pltpu.VMEM((K, LANE), jnp.int32),
            pltpu.VMEM((K, LANE), jnp.int32),
            pltpu.VMEM((K, LANE), jnp.int32),
            pltpu.VMEM((K, LANE), jnp.int32),
            pltpu.VMEM((K * LANE, HIDDEN), jnp.float32),
            pltpu.VMEM((K * LANE, HIDDEN), jnp.float32),
            pltpu.SemaphoreType.DMA((2,)),
            pltpu.SemaphoreType.DMA((2,)),
            pltpu.SemaphoreType.DMA((2,)),
        ],
    )
    def kern(e_hbm, x_hbm, out_hbm,
             acc, src0, src1, dst0, dst1, rows0, rows1, isem, gsem, ssem):
        src_hbm = e_hbm.at[0]
        dst_hbm = e_hbm.at[1]
        cid = lax.axis_index("c")
        sid = lax.axis_index("s")
        wid = cid * NS + sid

        srcb = [src0, src1]
        dstb = [dst0, dst1]
        rowsb = [rows0, rows1]

        zero16 = jnp.zeros((HIDDEN,), jnp.float32)

        @pl.loop(0, LANE)
        def _(i):
            rows0[i, :] = zero16

        zbase = sid * ROWS_PER_TILE

        @pl.loop(0, ROWS_PER_TILE // LANE)
        def _(b):
            pltpu.sync_copy(rows0.at[pl.ds(0, LANE), :],
                            acc.at[pl.ds(zbase + b * LANE, LANE), :])

        plsc.subcore_barrier()

        row0 = wid * (G * K)

        def idx_start(h, b):
            r = row0 + h * K
            pltpu.async_copy(src_hbm.at[pl.ds(r, K)], srcb[b], isem.at[b])
            pltpu.async_copy(dst_hbm.at[pl.ds(r, K)], dstb[b], isem.at[b])

        def idx_wait(b):
            pltpu.make_async_copy(src_hbm.at[pl.ds(0, K)], srcb[b],
                                  isem.at[b]).wait()
            pltpu.make_async_copy(dst_hbm.at[pl.ds(0, K)], dstb[b],
                                  isem.at[b]).wait()

        def gather_start(b):
            for j in range(K):
                pltpu.async_copy(x_hbm.at[srcb[b].at[j]],
                                 rowsb[b].at[pl.ds(j * LANE, LANE), :],
                                 gsem.at[b])

        def gather_wait(b):
            for j in range(K):
                pltpu.make_async_copy(x_hbm.at[srcb[b].at[j]],
                                      rowsb[b].at[pl.ds(j * LANE, LANE), :],
                                      gsem.at[b]).wait()

        def scatter_start(b):
            for j in range(K):
                pltpu.async_copy(rowsb[b].at[pl.ds(j * LANE, LANE), :],
                                 acc.at[dstb[b].at[j]], ssem.at[b], add=True)

        def scatter_wait(b):
            for j in range(K):
                pltpu.make_async_copy(rowsb[b].at[pl.ds(j * LANE, LANE), :],
                                      acc.at[dstb[b].at[j]],
                                      ssem.at[b]).wait()

        idx_start(0, 0)

        @pl.loop(0, G // 2)
        def _(p):
            g = 2 * p
            idx_wait(0)
            gather_start(0)

            @pl.when(p > 0)
            def _():
                scatter_wait(1)

            idx_start(g + 1, 1)
            gather_wait(0)
            scatter_start(0)

            idx_wait(1)
            gather_start(1)
            scatter_wait(0)

            @pl.when(p + 1 < G // 2)
            def _():
                idx_start(g + 2, 0)

            gather_wait(1)
            scatter_start(1)

        scatter_wait(1)

        plsc.subcore_barrier()

        @pl.loop(0, ROWS_PER_TILE // OCHUNK)
        def _(b):
            base = zbase + b * OCHUNK
            pltpu.sync_copy(acc.at[pl.ds(base, OCHUNK), :],
                            rows0.at[pl.ds(0, OCHUNK), :])
            pltpu.sync_copy(rows0.at[pl.ds(0, OCHUNK), :],
                            out_hbm.at[cid, pl.ds(base, OCHUNK), :])

    return kern(e3d, x_pad)


NPACK = 128 // HIDDEN
NROWS128 = N_PAD // NPACK
XB = 512


def _tc_pack_x(x):
    nblk = XB * NPACK

    def body(x_ref, o_ref):
        i = pl.program_id(0)
        xb = x_ref[...]
        node = i * nblk + lax.broadcasted_iota(jnp.int32, (nblk, 1), 0)
        valid = node < N_NODES
        packed = jnp.concatenate(
            [xb, jnp.ones((nblk, 1), jnp.float32),
             jnp.zeros((nblk, HIDDEN - IN_CH - 1), jnp.float32)], axis=1)
        packed = jnp.where(valid, packed, 0.0)
        o_ref[...] = jnp.concatenate(
            [packed[a * XB:(a + 1) * XB, :] for a in range(NPACK)], axis=1)

    return pl.pallas_call(
        body,
        grid=(NROWS128 // XB,),
        in_specs=[pl.BlockSpec((nblk, IN_CH), lambda i: (i, 0))],
        out_specs=pl.BlockSpec((XB, 128), lambda i: (i, 0)),
        out_shape=jax.ShapeDtypeStruct((NROWS128, 128), jnp.float32),
    )(x)


def _tc_combine(p128, x128, Wl128, Wr128, C128, b128):
    nb = NROWS128 // XB

    def body(p_ref, x_ref, wl_ref, wr_ref, c_ref, b_ref, o_ref):
        s = p_ref[0] + p_ref[1]
        cnt = jnp.maximum(
            jnp.dot(s, c_ref[...], preferred_element_type=jnp.float32), 1.0)
        agg = jnp.dot(s, wl_ref[...], preferred_element_type=jnp.float32) / cnt
        o = (
            agg
            + jnp.dot(x_ref[...], wr_ref[...],
                      preferred_element_type=jnp.float32)
            + b_ref[...]
        )
        o_ref[...] = jnp.concatenate(
            [o[:, HIDDEN * a:HIDDEN * (a + 1)] for a in range(NPACK)], axis=0)

    return pl.pallas_call(
        body,
        grid=(nb,),
        in_specs=[
            pl.BlockSpec((NC, XB, 128), lambda i: (0, i, 0)),
            pl.BlockSpec((XB, 128), lambda i: (i, 0)),
            pl.BlockSpec((128, 128), lambda i: (0, 0)),
            pl.BlockSpec((128, 128), lambda i: (0, 0)),
            pl.BlockSpec((128, 128), lambda i: (0, 0)),
            pl.BlockSpec((1, 128), lambda i: (0, 0)),
        ],
        out_specs=pl.BlockSpec((XB * NPACK, HIDDEN), lambda i: (i, 0)),
        out_shape=jax.ShapeDtypeStruct((N_NODES, HIDDEN), jnp.float32),
    )(p128, x128, Wl128, Wr128, C128, b128)


@jax.jit
def kernel(x, edge_index, W_l, W_r, b_l):
    idx = edge_index.astype(jnp.int32)
    n_edges = idx.shape[1]
    pad = EDGES_PAD - n_edges
    trash_rows = TRASH + jnp.arange(pad, dtype=jnp.int32) % (N_PAD - TRASH)
    e_pad = jnp.concatenate(
        [idx, jnp.broadcast_to(trash_rows, (2, pad))], axis=1)
    e_pad = ((e_pad & ~(XB * NPACK - 1))
             + ((e_pad & (XB - 1)) << 3)
             + ((e_pad & (XB * NPACK - 1)) >> 9))
    e3d = e_pad.reshape(2, NROWS, LANE)

    x128 = _tc_pack_x(x)
    partials = _sc_segment_sum(e3d, x128.reshape(N_PAD, HIDDEN))
    p128 = partials.reshape(NC, NROWS128, 128)

    eye8 = jnp.eye(NPACK, dtype=jnp.float32)
    Wl_pad = jnp.zeros((HIDDEN, HIDDEN), jnp.float32).at[:IN_CH].set(W_l)
    Wr_pad = jnp.zeros((HIDDEN, HIDDEN), jnp.float32).at[:IN_CH].set(W_r)
    Ci = jnp.zeros((HIDDEN, HIDDEN), jnp.float32).at[IN_CH].set(1.0)
    Wl128 = jnp.kron(eye8, Wl_pad)
    Wr128 = jnp.kron(eye8, Wr_pad)
    C128 = jnp.kron(eye8, Ci)
    b128 = jnp.tile(b_l, NPACK).reshape(1, 128)

    return _tc_combine(p128, x128, Wl128, Wr128, C128, b128)


# speedup vs baseline: 1.1495x; 1.0006x over previous
import functools

import jax
import jax.numpy as jnp
from jax import lax
from jax.experimental import pallas as pl
from jax.experimental.pallas import tpu as pltpu
from jax.experimental.pallas import tpu_sc as plsc

N_NODES = 100000
IN_CH = 10
HIDDEN = 16

NC = 2
NS = 16
NW = NC * NS
LANE = 128
K = 6
G = 262
EDGES_PAD = NW * G * K * LANE
NROWS = NW * G * K
ROWS_PER_TILE = 6400
N_PAD = NS * ROWS_PER_TILE
TRASH = N_NODES
OCHUNK = 640


def _sc_segment_sum(e3d, x_pad):
    mesh = plsc.VectorSubcoreMesh(core_axis_name="c", subcore_axis_name="s")

    @functools.partial(
        pl.kernel,
        out_type=jax.ShapeDtypeStruct((NC, N_PAD, HIDDEN), jnp.float32),
        mesh=mesh,
        compiler_params=pltpu.CompilerParams(use_tc_tiling_on_sc=False),
        scratch_types=[
            pltpu.VMEM_SHARED((N_PAD, HIDDEN), jnp.float32),
            pltpu.VMEM((K, LANE), jnp.int32),
            pltpu.VMEM((K, LANE), jnp.int32),
            pltpu.VMEM((K, LANE), jnp.int32),
            pltpu.VMEM((K, LANE), jnp.int32),
            pltpu.VMEM((K * LANE, HIDDEN), jnp.float32),
            pltpu.VMEM((K * LANE, HIDDEN), jnp.float32),
            pltpu.SemaphoreType.DMA((2,)),
            pltpu.SemaphoreType.DMA((2,)),
            pltpu.SemaphoreType.DMA((2,)),
        ],
    )
    def kern(e_hbm, x_hbm, out_hbm,
             acc, src0, src1, dst0, dst1, rows0, rows1, isem, gsem, ssem):
        cid = lax.axis_index("c")
        sid = lax.axis_index("s")
        wid = cid * NS + sid

        srcb = [src0, src1]
        dstb = [dst0, dst1]
        rowsb = [rows0, rows1]

        zero16 = jnp.zeros((HIDDEN,), jnp.float32)

        @pl.loop(0, LANE)
        def _(i):
            rows0[i, :] = zero16

        zbase = sid * ROWS_PER_TILE

        @pl.loop(0, ROWS_PER_TILE // LANE)
        def _(b):
            pltpu.sync_copy(rows0.at[pl.ds(0, LANE), :],
                            acc.at[pl.ds(zbase + b * LANE, LANE), :])

        plsc.subcore_barrier()

        row0 = wid * (G * K)

        def idx_start(h, b):
            r = row0 + h * K
            pltpu.async_copy(e_hbm.at[pl.ds(r, K)], srcb[b], isem.at[b])
            pltpu.async_copy(e_hbm.at[pl.ds(NROWS + r, K)], dstb[b],
                             isem.at[b])

        def idx_wait(b):
            pltpu.make_async_copy(e_hbm.at[pl.ds(0, K)], srcb[b],
                                  isem.at[b]).wait()
            pltpu.make_async_copy(e_hbm.at[pl.ds(0, K)], dstb[b],
                                  isem.at[b]).wait()

        def gather_start(b):
            for j in range(K):
                pltpu.async_copy(x_hbm.at[srcb[b].at[j]],
                                 rowsb[b].at[pl.ds(j * LANE, LANE), :],
                                 gsem.at[b])

        def gather_wait(b):
            for j in range(K):
                pltpu.make_async_copy(x_hbm.at[srcb[b].at[j]],
                                      rowsb[b].at[pl.ds(j * LANE, LANE), :],
                                      gsem.at[b]).wait()

        def scatter_start(b):
            for j in range(K):
                pltpu.async_copy(rowsb[b].at[pl.ds(j * LANE, LANE), :],
                                 acc.at[dstb[b].at[j]], ssem.at[b], add=True)

        def scatter_wait(b):
            for j in range(K):
                pltpu.make_async_copy(rowsb[b].at[pl.ds(j * LANE, LANE), :],
                                      acc.at[dstb[b].at[j]],
                                      ssem.at[b]).wait()

        idx_start(0, 0)

        @pl.loop(0, G // 2)
        def _(p):
            g = 2 * p
            idx_wait(0)
            gather_start(0)

            @pl.when(p > 0)
            def _():
                scatter_wait(1)

            idx_start(g + 1, 1)
            gather_wait(0)
            scatter_start(0)

            idx_wait(1)
            gather_start(1)
            scatter_wait(0)

            @pl.when(p + 1 < G // 2)
            def _():
                idx_start(g + 2, 0)

            gather_wait(1)
            scatter_start(1)

        scatter_wait(1)

        plsc.subcore_barrier()

        @pl.loop(0, ROWS_PER_TILE // OCHUNK)
        def _(b):
            base = zbase + b * OCHUNK
            pltpu.sync_copy(acc.at[pl.ds(base, OCHUNK), :],
                            rows0.at[pl.ds(0, OCHUNK), :])
            pltpu.sync_copy(rows0.at[pl.ds(0, OCHUNK), :],
                            out_hbm.at[cid, pl.ds(base, OCHUNK), :])

    return kern(e3d, x_pad)


NPACK = 128 // HIDDEN
NROWS128 = N_PAD // NPACK
XB = 512


def _tc_pack_x(x):
    nblk = XB * NPACK

    def body(x_ref, o_ref):
        i = pl.program_id(0)
        xb = x_ref[...]
        node = i * nblk + lax.broadcasted_iota(jnp.int32, (nblk, 1), 0)
        valid = node < N_NODES
        packed = jnp.concatenate(
            [xb, jnp.ones((nblk, 1), jnp.float32),
             jnp.zeros((nblk, HIDDEN - IN_CH - 1), jnp.float32)], axis=1)
        packed = jnp.where(valid, packed, 0.0)
        o_ref[...] = jnp.concatenate(
            [packed[a * XB:(a + 1) * XB, :] for a in range(NPACK)], axis=1)

    return pl.pallas_call(
        body,
        grid=(NROWS128 // XB,),
        in_specs=[pl.BlockSpec((nblk, IN_CH), lambda i: (i, 0))],
        out_specs=pl.BlockSpec((XB, 128), lambda i: (i, 0)),
        out_shape=jax.ShapeDtypeStruct((NROWS128, 128), jnp.float32),
    )(x)


def _tc_combine(p128, x128, Wl128, Wr128, C128, b128):
    nb = NROWS128 // XB

    def body(p_ref, x_ref, wl_ref, wr_ref, c_ref, b_ref, o_ref):
        s = p_ref[0] + p_ref[1]
        cnt = jnp.maximum(
            jnp.dot(s, c_ref[...], preferred_element_type=jnp.float32), 1.0)
        agg = jnp.dot(s, wl_ref[...], preferred_element_type=jnp.float32) / cnt
        o = (
            agg
            + jnp.dot(x_ref[...], wr_ref[...],
                      preferred_element_type=jnp.float32)
            + b_ref[...]
        )
        o_ref[...] = jnp.concatenate(
            [o[:, HIDDEN * a:HIDDEN * (a + 1)] for a in range(NPACK)], axis=0)

    return pl.pallas_call(
        body,
        grid=(nb,),
        in_specs=[
            pl.BlockSpec((NC, XB, 128), lambda i: (0, i, 0)),
            pl.BlockSpec((XB, 128), lambda i: (i, 0)),
            pl.BlockSpec((128, 128), lambda i: (0, 0)),
            pl.BlockSpec((128, 128), lambda i: (0, 0)),
            pl.BlockSpec((128, 128), lambda i: (0, 0)),
            pl.BlockSpec((1, 128), lambda i: (0, 0)),
        ],
        out_specs=pl.BlockSpec((XB * NPACK, HIDDEN), lambda i: (i, 0)),
        out_shape=jax.ShapeDtypeStruct((N_NODES, HIDDEN), jnp.float32),
    )(p128, x128, Wl128, Wr128, C128, b128)


@jax.jit
def kernel(x, edge_index, W_l, W_r, b_l):
    idx = edge_index.astype(jnp.int32)
    n_edges = idx.shape[1]
    pad = EDGES_PAD - n_edges
    trash_rows = TRASH + jnp.arange(pad, dtype=jnp.int32) % (N_PAD - TRASH)
    e_pad = jnp.concatenate(
        [idx, jnp.broadcast_to(trash_rows, (2, pad))], axis=1)
    e_pad = ((e_pad & ~(XB * NPACK - 1))
             + ((e_pad & (XB - 1)) << 3)
             + ((e_pad & (XB * NPACK - 1)) >> 9))
    e3d = e_pad.reshape(2, NROWS, LANE).reshape(2 * NROWS, LANE)

    x128 = _tc_pack_x(x)
    partials = _sc_segment_sum(e3d, x128.reshape(N_PAD, HIDDEN))
    p128 = partials.reshape(NC, NROWS128, 128)

    eye8 = jnp.eye(NPACK, dtype=jnp.float32)
    Wl_pad = jnp.zeros((HIDDEN, HIDDEN), jnp.float32).at[:IN_CH].set(W_l)
    Wr_pad = jnp.zeros((HIDDEN, HIDDEN), jnp.float32).at[:IN_CH].set(W_r)
    Ci = jnp.zeros((HIDDEN, HIDDEN), jnp.float32).at[IN_CH].set(1.0)
    Wl128 = jnp.kron(eye8, Wl_pad)
    Wr128 = jnp.kron(eye8, Wr_pad)
    C128 = jnp.kron(eye8, Ci)
    b128 = jnp.tile(b_l, NPACK).reshape(1, 128)

    return _tc_combine(p128, x128, Wl128, Wr128, C128, b128)
